# Initial kernel scaffold; baseline (speedup 1.0000x reference)
import jax
import jax.numpy as jnp
from jax.experimental import pallas as pl


def kernel(h_e, edge_input, node_input, speed_kph, lanes, length, origin_in_degree, origin_out_degree, dest_in_degree, dest_out_degree, lat, lon, inputLanes, outputLanes, W_n1, b_n1, W_n2, b_n2, W_r1, b_r1, W_r2, b_r2, W_e1, b_e1, W_e2, b_e2, edge_index):
    raise NotImplementedError("write your pallas kernel here")


# speedup vs baseline: 8.0506x; 8.0506x over previous
import jax
import jax.numpy as jnp
from jax import lax
from jax.experimental import pallas as pl
from jax.experimental.pallas import tpu as pltpu
from jax.experimental.pallas import tpu_sc as plsc

_N = 10000
_E = 320000
_F = 128

_NC = 2
_NS = 16
_NW = _NC * _NS
_EPW = _E // _NW
_CH = 80
_NCH = _EPW // _CH
_RPW = 624
_RTAIL = _N - _NS * _RPW


def _segsum_body(v_hbm, dst_hbm, zero_hbm, out_hbm, idx_v, rows_v, acc_sh):
    c = lax.axis_index("c")
    s = lax.axis_index("s")
    wid = s * _NC + c

    @pl.when(s == 0)
    def _init():
        pltpu.sync_copy(zero_hbm, acc_sh)

    plsc.subcore_barrier()

    def body(i, carry):
        base = wid * _EPW + i * _CH
        pltpu.sync_copy(dst_hbm.at[pl.ds(base, _CH)], idx_v)
        pltpu.sync_copy(v_hbm.at[pl.ds(base, _CH)], rows_v)
        pltpu.sync_copy(rows_v, acc_sh.at[idx_v], add=True)
        return carry

    lax.fori_loop(0, _NCH, body, 0)
    plsc.subcore_barrier()
    pltpu.sync_copy(acc_sh.at[pl.ds(s * _RPW, _RPW)],
                    out_hbm.at[pl.ds(c * _N + s * _RPW, _RPW)])

    @pl.when(s == 0)
    def _tail():
        pltpu.sync_copy(acc_sh.at[pl.ds(_NS * _RPW, _RTAIL)],
                        out_hbm.at[pl.ds(c * _N + _NS * _RPW, _RTAIL)])


_segsum_call = pl.kernel(
    _segsum_body,
    out_type=jax.ShapeDtypeStruct((2 * _N, _F), jnp.float32),
    mesh=plsc.VectorSubcoreMesh(core_axis_name="c", subcore_axis_name="s"),
    scratch_types=[pltpu.VMEM((_CH,), jnp.int32),
                   pltpu.VMEM((_CH, _F), jnp.float32),
                   pltpu.VMEM_SHARED((_N, _F), jnp.float32)],
)


def _make_gather_pair(d):
    def body(a_hbm, b_hbm, src_hbm, dst_hbm, sa_hbm, sb_hbm,
             idxs_v, idxd_v, bufa_v, bufb_v, sema, semb):
        c = lax.axis_index("c")
        s = lax.axis_index("s")
        wid = s * _NC + c

        def step(i, carry):
            base = wid * _EPW + i * _CH
            pltpu.sync_copy(src_hbm.at[pl.ds(base, _CH)], idxs_v)
            pltpu.sync_copy(dst_hbm.at[pl.ds(base, _CH)], idxd_v)
            cpa = pltpu.async_copy(a_hbm.at[idxs_v], bufa_v, sema)
            cpb = pltpu.async_copy(b_hbm.at[idxd_v], bufb_v, semb)
            cpa.wait()
            cpb.wait()
            pltpu.sync_copy(bufa_v, sa_hbm.at[pl.ds(base, _CH)])
            pltpu.sync_copy(bufb_v, sb_hbm.at[pl.ds(base, _CH)])
            return carry

        lax.fori_loop(0, _NCH, step, 0)

    return pl.kernel(
        body,
        out_type=[jax.ShapeDtypeStruct((_E, d), jnp.float32),
                  jax.ShapeDtypeStruct((_E, d), jnp.float32)],
        mesh=plsc.VectorSubcoreMesh(core_axis_name="c", subcore_axis_name="s"),
        scratch_types=[pltpu.VMEM((_CH,), jnp.int32),
                       pltpu.VMEM((_CH,), jnp.int32),
                       pltpu.VMEM((_CH, d), jnp.float32),
                       pltpu.VMEM((_CH, d), jnp.float32),
                       pltpu.SemaphoreType.DMA,
                       pltpu.SemaphoreType.DMA],
    )


_gather_pair_128 = _make_gather_pair(_F)


_BE = 2560
_BN = 2000


def _prep_body(nrest_ref, wsel_ref, p_ref, pneg_ref):
    p = jnp.dot(nrest_ref[...], wsel_ref[...],
                preferred_element_type=jnp.float32)
    p_ref[...] = p
    pneg_ref[...] = -p


def _prep(nrest, wsel):
    return pl.pallas_call(
        _prep_body,
        grid=(_N // _BN,),
        in_specs=[
            pl.BlockSpec((_BN, 32), lambda i: (i, 0)),
            pl.BlockSpec((32, _F), lambda i: (0, 0)),
        ],
        out_specs=[
            pl.BlockSpec((_BN, _F), lambda i: (i, 0)),
            pl.BlockSpec((_BN, _F), lambda i: (i, 0)),
        ],
        out_shape=[
            jax.ShapeDtypeStruct((_N, _F), jnp.float32),
            jax.ShapeDtypeStruct((_N, _F), jnp.float32),
        ],
    )(nrest, wsel)


def _edge_mlp1_body(he_ref, rest_ref, gs_ref, gd_ref, wh_ref, wr_ref,
                    b1_ref, w2_ref, b2_ref, out_ref):
    x = jnp.dot(he_ref[...], wh_ref[...], preferred_element_type=jnp.float32)
    x = x + jnp.dot(rest_ref[...], wr_ref[...],
                    preferred_element_type=jnp.float32)
    x = x + gs_ref[...] + gd_ref[...]
    h = jnp.maximum(x + b1_ref[...], 0.0)
    out_ref[...] = jnp.dot(h, w2_ref[...],
                           preferred_element_type=jnp.float32) + b2_ref[...]


def _edge_mlp1(h_e, rest, gs, gd, wh, wr, b1, w2, b2):
    return pl.pallas_call(
        _edge_mlp1_body,
        grid=(_E // _BE,),
        in_specs=[
            pl.BlockSpec((_BE, _F), lambda i: (i, 0)),
            pl.BlockSpec((_BE, 32), lambda i: (i, 0)),
            pl.BlockSpec((_BE, _F), lambda i: (i, 0)),
            pl.BlockSpec((_BE, _F), lambda i: (i, 0)),
            pl.BlockSpec((_F, _F), lambda i: (0, 0)),
            pl.BlockSpec((32, _F), lambda i: (0, 0)),
            pl.BlockSpec((1, _F), lambda i: (0, 0)),
            pl.BlockSpec((_F, _F), lambda i: (0, 0)),
            pl.BlockSpec((1, _F), lambda i: (0, 0)),
        ],
        out_specs=pl.BlockSpec((_BE, _F), lambda i: (i, 0)),
        out_shape=jax.ShapeDtypeStruct((_E, _F), jnp.float32),
    )(h_e, rest, gs, gd, wh, wr, b1, w2, b2)


def _node_body(p0_ref, p1_ref, nrest_ref, ws_ref, wnr_ref, b1_ref, w2_ref,
               b2_ref, wu_ref, wv_ref, wsel_ref, s_ref, score_ref, a_ref,
               b_ref):
    sblk = p0_ref[...] + p1_ref[...]
    s_ref[...] = sblk
    x = jnp.dot(sblk, ws_ref[...], preferred_element_type=jnp.float32)
    x = x + jnp.dot(nrest_ref[...], wnr_ref[...],
                    preferred_element_type=jnp.float32)
    h = jnp.maximum(x + b1_ref[...], 0.0)
    sc = jnp.dot(h, w2_ref[...], preferred_element_type=jnp.float32) + b2_ref[...]
    score_ref[...] = sc
    pe = jnp.dot(nrest_ref[...], wsel_ref[...],
                 preferred_element_type=jnp.float32)
    a_ref[...] = jnp.dot(sc, wu_ref[...],
                         preferred_element_type=jnp.float32) + pe
    b_ref[...] = jnp.dot(sc, wv_ref[...],
                         preferred_element_type=jnp.float32) - pe


def _node_mlp(partials, nrest, ws, wnr, b1, w2, b2, wu, wv, wsel):
    nb = _N // _BN
    return pl.pallas_call(
        _node_body,
        grid=(nb,),
        in_specs=[
            pl.BlockSpec((_BN, _F), lambda i: (i, 0)),
            pl.BlockSpec((_BN, _F), lambda i: (i + _N // _BN, 0)),
            pl.BlockSpec((_BN, 32), lambda i: (i, 0)),
            pl.BlockSpec((_F, _F), lambda i: (0, 0)),
            pl.BlockSpec((32, _F), lambda i: (0, 0)),
            pl.BlockSpec((1, _F), lambda i: (0, 0)),
            pl.BlockSpec((_F, _F), lambda i: (0, 0)),
            pl.BlockSpec((1, _F), lambda i: (0, 0)),
            pl.BlockSpec((_F, _F), lambda i: (0, 0)),
            pl.BlockSpec((_F, _F), lambda i: (0, 0)),
            pl.BlockSpec((32, _F), lambda i: (0, 0)),
        ],
        out_specs=[
            pl.BlockSpec((_BN, _F), lambda i: (i, 0)),
            pl.BlockSpec((_BN, _F), lambda i: (i, 0)),
            pl.BlockSpec((_BN, _F), lambda i: (i, 0)),
            pl.BlockSpec((_BN, _F), lambda i: (i, 0)),
        ],
        out_shape=[
            jax.ShapeDtypeStruct((_N, _F), jnp.float32),
            jax.ShapeDtypeStruct((_N, _F), jnp.float32),
            jax.ShapeDtypeStruct((_N, _F), jnp.float32),
            jax.ShapeDtypeStruct((_N, _F), jnp.float32),
        ],
    )(partials, partials, nrest, ws, wnr, b1, w2, b2, wu, wv, wsel)


def _edge_mlp2_body(sa_ref, sb_ref, rest_ref, wr_ref, b1_ref, w2_ref, b2_ref,
                    out_ref):
    x = sa_ref[...] + sb_ref[...]
    x = x + jnp.dot(rest_ref[...], wr_ref[...],
                    preferred_element_type=jnp.float32)
    h = jnp.maximum(x + b1_ref[...], 0.0)
    out_ref[...] = jnp.dot(h, w2_ref[...],
                           preferred_element_type=jnp.float32) + b2_ref[...]


def _edge_mlp2(sa, sb, rest, wr, b1, w2, b2):
    return pl.pallas_call(
        _edge_mlp2_body,
        grid=(_E // _BE,),
        in_specs=[
            pl.BlockSpec((_BE, _F), lambda i: (i, 0)),
            pl.BlockSpec((_BE, _F), lambda i: (i, 0)),
            pl.BlockSpec((_BE, 32), lambda i: (i, 0)),
            pl.BlockSpec((32, _F), lambda i: (0, 0)),
            pl.BlockSpec((1, _F), lambda i: (0, 0)),
            pl.BlockSpec((_F, _F), lambda i: (0, 0)),
            pl.BlockSpec((1, _F), lambda i: (0, 0)),
        ],
        out_specs=pl.BlockSpec((_BE, _F), lambda i: (i, 0)),
        out_shape=jax.ShapeDtypeStruct((_E, _F), jnp.float32),
    )(sa, sb, rest, wr, b1, w2, b2)


def kernel(h_e, edge_input, node_input, speed_kph, lanes, length,
           origin_in_degree, origin_out_degree, dest_in_degree,
           dest_out_degree, lat, lon, inputLanes, outputLanes,
           W_n1, b_n1, W_n2, b_n2, W_r1, b_r1, W_r2, b_r2,
           W_e1, b_e1, W_e2, b_e2, edge_index):
    src = edge_index[0]
    dst = edge_index[1]

    zc2 = jnp.zeros((_E, 2), jnp.float32)
    rest = jnp.concatenate([
        speed_kph[:, None], lanes[:, None], length[:, None],
        zc2,
        origin_in_degree[:, None], origin_out_degree[:, None],
        dest_in_degree[:, None], dest_out_degree[:, None],
        edge_input,
        jnp.zeros((_E, 7), jnp.float32),
    ], axis=1)

    nrest = jnp.concatenate([
        node_input, lat[:, None], lon[:, None],
        inputLanes[:, None], outputLanes[:, None],
        jnp.zeros((_N, 12), jnp.float32),
    ], axis=1)

    zpad7 = jnp.zeros((_F, 7), jnp.float32)
    zpad12 = jnp.zeros((_F, 12), jnp.float32)

    def _latlon_sel(w_lat, w_lon):
        wsel = jnp.zeros((32, _F), jnp.float32)
        return wsel.at[16].set(w_lat).at[17].set(w_lon)

    pn, pn_neg = _prep(nrest, _latlon_sel(W_n1[:, 131], W_n1[:, 132]))
    gs, gd = _gather_pair_128(pn, pn_neg, src, dst)

    wh = W_n1[:, :_F].T
    wr_n = jnp.concatenate([W_n1[:, _F:], zpad7], axis=1).T
    v = _edge_mlp1(h_e, rest, gs, gd, wh, wr_n, b_n1[None], W_n2.T,
                   b_n2[None])

    partials = _segsum_call(v, dst, jnp.zeros((_N, _F), jnp.float32))

    ws = W_r1[:, :_F].T
    wnr = jnp.concatenate([W_r1[:, _F:], zpad12], axis=1).T
    wu = W_e1[:, :_F].T
    wv = W_e1[:, _F:2 * _F].T
    wsel_e = _latlon_sel(W_e1[:, 259], W_e1[:, 260])
    s_sum, score, a, b = _node_mlp(partials, nrest, ws, wnr, b_r1[None],
                                   W_r2.T, b_r2[None], wu, wv, wsel_e)

    sa, sb = _gather_pair_128(a, b, src, dst)

    wer = jnp.concatenate([W_e1[:, 2 * _F:], zpad7], axis=1).T
    h_e_new = _edge_mlp2(sa, sb, rest, wer, b_e1[None], W_e2.T, b_e2[None])

    return (h_e_new, score, s_sum)
